# Initial kernel scaffold; baseline (speedup 1.0000x reference)
import jax
import jax.numpy as jnp
from jax.experimental import pallas as pl


def kernel(queries, pos_emb, lvl_emb, cam_emb, feat0, reference_points_3D, bev_mask, W_so, b_so, W_aw, b_aw, W_v, b_v, W_out, b_out):
    raise NotImplementedError("write your pallas kernel here")


# speedup vs baseline: 89.8292x; 89.8292x over previous
import functools

import jax
import jax.numpy as jnp
from jax import lax
from jax.experimental import pallas as pl
from jax.experimental.pallas import tpu as pltpu
from jax.experimental.pallas import tpu_sc as plsc

N_CAM = 6
Q_LEN = 10000
D = 256
H = 8
DH = 32
P = 8
NUM_Z = 4
FH = 60
FW = 100
S = FH * FW
ROWS = N_CAM * Q_LEN
NSLOT = H * P * 4

NPTS = H * P
NW = 32
CH = 8
NCHUNK = ROWS // CH
GSLOTS = CH * NPTS


def _tc_value_table(feat2, lvl_emb, cam_emb, wv3, bv2):
    def body(f_ref, lvl_ref, cam_ref, wv_ref, bv_ref, out_ref):
        f = f_ref[0]
        wv = wv_ref[0]
        t = lax.dot_general(f, wv, (((0,), (1,)), ((), ())),
                            preferred_element_type=jnp.float32,
                            precision=lax.Precision.HIGHEST)
        lc = lvl_ref[...] + cam_ref[0]
        rb = lax.dot_general(lc, wv, (((1,), (1,)), ((), ())),
                             preferred_element_type=jnp.float32,
                            precision=lax.Precision.HIGHEST)
        t = t + rb + bv_ref[0]
        z = jnp.zeros((101, DH), jnp.float32)
        t1 = jnp.concatenate([t[1:], z[:1]], axis=0)
        t100 = jnp.concatenate([t[FW:], z[:FW]], axis=0)
        t101 = jnp.concatenate([t[FW + 1:], z], axis=0)
        out_ref[0] = jnp.concatenate([t, t1, t100, t101], axis=1)

    return pl.pallas_call(
        body,
        grid=(N_CAM, H),
        in_specs=[
            pl.BlockSpec((1, D, S), lambda n, h: (n, 0, 0)),
            pl.BlockSpec((1, D), lambda n, h: (0, 0)),
            pl.BlockSpec((1, 1, D), lambda n, h: (n, 0, 0)),
            pl.BlockSpec((1, DH, D), lambda n, h: (h, 0, 0)),
            pl.BlockSpec((1, 1, DH), lambda n, h: (h, 0, 0)),
        ],
        out_specs=pl.BlockSpec((1, S, 4 * DH), lambda n, h: (n * H + h, 0, 0)),
        out_shape=jax.ShapeDtypeStruct((N_CAM * H, S, 4 * DH), jnp.float32),
    )(feat2, lvl_emb, cam_emb, wv3, bv2)


BQ = 1000
NQB = Q_LEN // BQ


def _tc_index_weights(q2, p2, rpt2, bm2, wso_t, bso2, waw_t, baw2):
    def body(q_ref, p_ref, rpt_ref, bm_ref, wso_ref, bso_ref, waw_ref,
             baw_ref, idx_ref, w_ref):
        n = pl.program_id(0)
        qr = q_ref[0] + p_ref[0]
        so = lax.dot_general(qr, wso_ref[...], (((1,), (0,)), ((), ())),
                             preferred_element_type=jnp.float32,
                            precision=lax.Precision.HIGHEST) + bso_ref[...]
        awl = lax.dot_general(qr, waw_ref[...], (((1,), (0,)), ((), ())),
                              preferred_element_type=jnp.float32,
                            precision=lax.Precision.HIGHEST) + baw_ref[...]
        m = jnp.max(awl, axis=1, keepdims=True)
        e = jnp.exp(awl - m)
        gi = lax.broadcasted_iota(jnp.int32, (64, 64), 0) // P
        gj = lax.broadcasted_iota(jnp.int32, (64, 64), 1) // P
        G = (gi == gj).astype(jnp.float32)
        ssum = lax.dot_general(e, G, (((1,), (0,)), ((), ())),
                               preferred_element_type=jnp.float32,
                            precision=lax.Precision.HIGHEST)
        aw = e / ssum

        zi = lax.broadcasted_iota(jnp.int32, (P, 1), 0)
        selz = (zi % 2 == 0).astype(jnp.float32)
        act = (lax.dot_general(bm_ref[0], selz, (((1,), (0,)), ((), ())),
                               preferred_element_type=jnp.float32,
                            precision=lax.Precision.HIGHEST)
               > 0).astype(jnp.float32)
        awa = aw * act

        rj = lax.broadcasted_iota(jnp.int32, (P, 64), 0)
        cz = lax.broadcasted_iota(jnp.int32, (P, 64), 1) % NUM_Z
        Sx = (rj == 2 * cz).astype(jnp.float32)
        Sy = (rj == 2 * cz + 1).astype(jnp.float32)
        rx = lax.dot_general(rpt_ref[0], Sx, (((1,), (0,)), ((), ())),
                             preferred_element_type=jnp.float32,
                            precision=lax.Precision.HIGHEST)
        ry = lax.dot_general(rpt_ref[0], Sy, (((1,), (0,)), ((), ())),
                             preferred_element_type=jnp.float32,
                            precision=lax.Precision.HIGHEST)

        sox = so[:, :64]
        soy = so[:, 64:]
        x = rx * FW + sox - 0.5
        y = ry * FH + soy - 0.5
        x0 = jnp.floor(x)
        y0 = jnp.floor(y)
        fx = x - x0
        fy = y - y0

        hcol = lax.broadcasted_iota(jnp.int32, (BQ, 64), 1) // P
        base = n * (H * S) + hcol * S

        xb = jnp.clip(x0, 0.0, FW - 2)
        yb = jnp.clip(y0, 0.0, FH - 2)
        idx_ref[...] = (base + yb.astype(jnp.int32) * FW
                        + xb.astype(jnp.int32))

        w_parts = []
        for dy in (0, 1):
            for dx in (0, 1):
                xs = xb + dx
                ys = yb + dy
                wx = (jnp.where(xs == x0, 1.0 - fx, 0.0)
                      + jnp.where(xs == x0 + 1.0, fx, 0.0))
                wy = (jnp.where(ys == y0, 1.0 - fy, 0.0)
                      + jnp.where(ys == y0 + 1.0, fy, 0.0))
                w_parts.append(awa * wx * wy)
        w_ref[...] = jnp.concatenate(w_parts, axis=1)

    return pl.pallas_call(
        body,
        grid=(N_CAM, NQB),
        in_specs=[
            pl.BlockSpec((1, BQ, D), lambda n, qb: (n, qb, 0)),
            pl.BlockSpec((1, BQ, D), lambda n, qb: (n, qb, 0)),
            pl.BlockSpec((1, BQ, P), lambda n, qb: (n, qb, 0)),
            pl.BlockSpec((1, BQ, P), lambda n, qb: (n, qb, 0)),
            pl.BlockSpec((D, 2 * H * P), lambda n, qb: (0, 0)),
            pl.BlockSpec((1, 2 * H * P), lambda n, qb: (0, 0)),
            pl.BlockSpec((D, H * P), lambda n, qb: (0, 0)),
            pl.BlockSpec((1, H * P), lambda n, qb: (0, 0)),
        ],
        out_specs=[
            pl.BlockSpec((BQ, NPTS), lambda n, qb: (n * NQB + qb, 0)),
            pl.BlockSpec((BQ, NSLOT), lambda n, qb: (n * NQB + qb, 0)),
        ],
        out_shape=[
            jax.ShapeDtypeStruct((ROWS, NPTS), jnp.int32),
            jax.ShapeDtypeStruct((ROWS, NSLOT), jnp.float32),
        ],
    )(q2, p2, rpt2, bm2, wso_t, bso2, waw_t, baw2)


def _broadcast_lane(vec16, lane):
    idx = jnp.full((16, 1), lane, dtype=jnp.int32)
    dn = lax.GatherDimensionNumbers(offset_dims=(), collapsed_slice_dims=(0,),
                                    start_index_map=(0,))
    return lax.gather(vec16, idx, dn, (1,),
                      mode=lax.GatherScatterMode.PROMISE_IN_BOUNDS)


def _sc_gather_accum(table, idx2, w2):
    mesh = plsc.VectorSubcoreMesh(core_axis_name="c", subcore_axis_name="s")

    @functools.partial(
        pl.kernel,
        mesh=mesh,
        out_type=jax.ShapeDtypeStruct((ROWS, NSLOT), jnp.float32),
        scratch_types=[
            pltpu.VMEM((CH, NPTS), jnp.int32),
            pltpu.VMEM((2 * CH, 128), jnp.float32),
            pltpu.VMEM((GSLOTS, 4 * DH), jnp.float32),
            pltpu.VMEM((CH, NSLOT), jnp.float32),
            pltpu.SemaphoreType.DMA,
        ],
    )
    def sc_kernel(table_hbm, idx_hbm, w_hbm, out_hbm, idx_v, w_v, rows_v,
                  out_v, gsem):
        wid = lax.axis_index("s") * 2 + lax.axis_index("c")
        nbase = NCHUNK // NW
        nch = nbase + (wid < NCHUNK - nbase * NW).astype(jnp.int32)

        def chunk_body(i, _):
            c = wid + i * NW
            base = c * CH
            pltpu.sync_copy(idx_hbm.at[pl.ds(base, CH), :], idx_v)
            pltpu.sync_copy(w_hbm.at[pl.ds(2 * base, 2 * CH), :], w_v)
            copies = [
                pltpu.async_copy(table_hbm.at[idx_v.at[j]],
                                 rows_v.at[pl.ds(j * NPTS, NPTS)], gsem)
                for j in range(CH)
            ]
            for cp in copies:
                cp.wait()

            def group_body(g, _):
                r = g // H
                h = g - r * H
                acc0 = jnp.zeros((16,), jnp.float32)
                acc1 = jnp.zeros((16,), jnp.float32)
                for sl in range(4):
                    wrow = 2 * r + (sl // 2)
                    astart = (sl % 2) * 64 + (h // 2) * 16
                    wreg = w_v[wrow, pl.ds(astart, 16)]
                    lane0 = (h % 2) * 8
                    for p in range(P):
                        wb = _broadcast_lane(wreg, lane0 + p)
                        g_row = r * NPTS + h * P + p
                        acc0 = acc0 + wb * rows_v[g_row, pl.ds(sl * DH, 16)]
                        acc1 = acc1 + wb * rows_v[g_row,
                                                  pl.ds(sl * DH + 16, 16)]
                out_v[r, pl.ds(h * DH, 16)] = acc0
                out_v[r, pl.ds(h * DH + 16, 16)] = acc1
                return 0

            lax.fori_loop(0, CH * H, group_body, 0)
            pltpu.sync_copy(out_v, out_hbm.at[pl.ds(base, CH), :])
            return 0

        lax.fori_loop(0, nch, chunk_body, 0)

    return sc_kernel(table, idx2, w2)


def _tc_reduce_project(sc_out3, bm2, wout_t, bout2):
    def body(s_ref, bm_ref, w_ref, b_ref, o_ref):
        acc = s_ref[0]
        for i in range(1, N_CAM):
            acc = acc + s_ref[i]
        zi = lax.broadcasted_iota(jnp.int32, (P, 1), 0)
        selz = (zi % 2 == 0).astype(jnp.float32)
        cnt = jnp.zeros((BQ, 1), jnp.float32)
        for i in range(N_CAM):
            cnt = cnt + (lax.dot_general(
                bm_ref[i], selz, (((1,), (0,)), ((), ())),
                preferred_element_type=jnp.float32,
                            precision=lax.Precision.HIGHEST) > 0).astype(jnp.float32)
        slots = acc / jnp.maximum(cnt, 1.0)
        o_ref[0] = lax.dot_general(slots, w_ref[...], (((1,), (0,)), ((), ())),
                                   preferred_element_type=jnp.float32,
                            precision=lax.Precision.HIGHEST) + b_ref[...]

    return pl.pallas_call(
        body,
        grid=(NQB,),
        in_specs=[
            pl.BlockSpec((N_CAM, BQ, D), lambda qb: (0, qb, 0)),
            pl.BlockSpec((N_CAM, BQ, P), lambda qb: (0, qb, 0)),
            pl.BlockSpec((D, D), lambda qb: (0, 0)),
            pl.BlockSpec((1, D), lambda qb: (0, 0)),
        ],
        out_specs=pl.BlockSpec((1, BQ, D), lambda qb: (0, qb, 0)),
        out_shape=jax.ShapeDtypeStruct((1, Q_LEN, D), jnp.float32),
    )(sc_out3, bm2, wout_t, bout2)


def kernel(queries, pos_emb, lvl_emb, cam_emb, feat0, reference_points_3D,
           bev_mask, W_so, b_so, W_aw, b_aw, W_v, b_v, W_out, b_out):
    feat2 = feat0.reshape(N_CAM, D, S)
    cam3 = cam_emb.reshape(N_CAM, 1, D)
    wv3 = W_v.reshape(H, DH, D)
    bv3 = b_v.reshape(H, 1, DH)
    table = _tc_value_table(feat2, lvl_emb, cam3, wv3, bv3)
    table = table.reshape(N_CAM * H * S, 4 * DH)

    W_so_x = W_so[0::2]
    W_so_y = W_so[1::2]
    wso_t = jnp.concatenate([W_so_x, W_so_y], axis=0).T
    bso2 = jnp.concatenate([b_so[0::2], b_so[1::2]])[None, :]
    waw_t = W_aw.T
    baw2 = b_aw[None, :]

    q2 = queries.reshape(N_CAM, Q_LEN, D)
    p2 = pos_emb.reshape(N_CAM, Q_LEN, D)
    rpt2 = reference_points_3D.reshape(N_CAM, Q_LEN, NUM_Z * 2)
    bm2 = bev_mask.reshape(N_CAM, Q_LEN, NUM_Z * 2).astype(jnp.float32)

    idx, w = _tc_index_weights(q2, p2, rpt2, bm2, wso_t, bso2, waw_t, baw2)

    sc_out = _sc_gather_accum(table, idx, w.reshape(ROWS * 2, 128))
    sc_out3 = sc_out.reshape(N_CAM, Q_LEN, D)

    return _tc_reduce_project(sc_out3, bm2, W_out.T, b_out[None, :])


# speedup vs baseline: 115.7893x; 1.2890x over previous
import functools

import jax
import jax.numpy as jnp
from jax import lax
from jax.experimental import pallas as pl
from jax.experimental.pallas import tpu as pltpu
from jax.experimental.pallas import tpu_sc as plsc

N_CAM = 6
Q_LEN = 10000
D = 256
H = 8
DH = 32
P = 8
NUM_Z = 4
FH = 60
FW = 100
S = FH * FW
ROWS = N_CAM * Q_LEN
NSLOT = H * P * 4

NPTS = H * P
NW = 32
CH = 8
NCHUNK = ROWS // CH
GSLOTS = CH * NPTS


def _tc_value_table(feat2, lvl_emb, cam_emb, wv3, bv2):
    def body(f_ref, lvl_ref, cam_ref, wv_ref, bv_ref, out_ref):
        f = f_ref[0]
        wv = wv_ref[0]
        t = lax.dot_general(f, wv, (((0,), (1,)), ((), ())),
                            preferred_element_type=jnp.float32,
                            precision=lax.Precision.HIGHEST)
        lc = lvl_ref[...] + cam_ref[0]
        rb = lax.dot_general(lc, wv, (((1,), (1,)), ((), ())),
                             preferred_element_type=jnp.float32,
                            precision=lax.Precision.HIGHEST)
        t = t + rb + bv_ref[0]
        z = jnp.zeros((101, DH), jnp.float32)
        t1 = jnp.concatenate([t[1:], z[:1]], axis=0)
        t100 = jnp.concatenate([t[FW:], z[:FW]], axis=0)
        t101 = jnp.concatenate([t[FW + 1:], z], axis=0)
        out_ref[0] = jnp.concatenate([t, t1, t100, t101], axis=1)

    return pl.pallas_call(
        body,
        grid=(N_CAM, H),
        in_specs=[
            pl.BlockSpec((1, D, S), lambda n, h: (n, 0, 0)),
            pl.BlockSpec((1, D), lambda n, h: (0, 0)),
            pl.BlockSpec((1, 1, D), lambda n, h: (n, 0, 0)),
            pl.BlockSpec((1, DH, D), lambda n, h: (h, 0, 0)),
            pl.BlockSpec((1, 1, DH), lambda n, h: (h, 0, 0)),
        ],
        out_specs=pl.BlockSpec((1, S, 4 * DH), lambda n, h: (n * H + h, 0, 0)),
        out_shape=jax.ShapeDtypeStruct((N_CAM * H, S, 4 * DH), jnp.float32),
    )(feat2, lvl_emb, cam_emb, wv3, bv2)


BQ = 1000
NQB = Q_LEN // BQ


def _tc_index_weights(q2, p2, rpt2, bm2, wso_t, bso2, waw_t, baw2):
    def body(q_ref, p_ref, rpt_ref, bm_ref, wso_ref, bso_ref, waw_ref,
             baw_ref, idx_ref, w_ref):
        n = pl.program_id(0)
        qr = q_ref[0] + p_ref[0]
        so = lax.dot_general(qr, wso_ref[...], (((1,), (0,)), ((), ())),
                             preferred_element_type=jnp.float32,
                            precision=lax.Precision.HIGHEST) + bso_ref[...]
        awl = lax.dot_general(qr, waw_ref[...], (((1,), (0,)), ((), ())),
                              preferred_element_type=jnp.float32,
                            precision=lax.Precision.HIGHEST) + baw_ref[...]
        m = jnp.max(awl, axis=1, keepdims=True)
        e = jnp.exp(awl - m)
        gi = lax.broadcasted_iota(jnp.int32, (64, 64), 0) // P
        gj = lax.broadcasted_iota(jnp.int32, (64, 64), 1) // P
        G = (gi == gj).astype(jnp.float32)
        ssum = lax.dot_general(e, G, (((1,), (0,)), ((), ())),
                               preferred_element_type=jnp.float32,
                            precision=lax.Precision.HIGHEST)
        aw = e / ssum

        zi = lax.broadcasted_iota(jnp.int32, (P, 1), 0)
        selz = (zi % 2 == 0).astype(jnp.float32)
        act = (lax.dot_general(bm_ref[0], selz, (((1,), (0,)), ((), ())),
                               preferred_element_type=jnp.float32,
                            precision=lax.Precision.HIGHEST)
               > 0).astype(jnp.float32)
        awa = aw * act

        rj = lax.broadcasted_iota(jnp.int32, (P, 64), 0)
        cz = lax.broadcasted_iota(jnp.int32, (P, 64), 1) % NUM_Z
        Sx = (rj == 2 * cz).astype(jnp.float32)
        Sy = (rj == 2 * cz + 1).astype(jnp.float32)
        rx = lax.dot_general(rpt_ref[0], Sx, (((1,), (0,)), ((), ())),
                             preferred_element_type=jnp.float32,
                            precision=lax.Precision.HIGHEST)
        ry = lax.dot_general(rpt_ref[0], Sy, (((1,), (0,)), ((), ())),
                             preferred_element_type=jnp.float32,
                            precision=lax.Precision.HIGHEST)

        sox = so[:, :64]
        soy = so[:, 64:]
        x = rx * FW + sox - 0.5
        y = ry * FH + soy - 0.5
        x0 = jnp.floor(x)
        y0 = jnp.floor(y)
        fx = x - x0
        fy = y - y0

        hcol = lax.broadcasted_iota(jnp.int32, (BQ, 64), 1) // P
        base = n * (H * S) + hcol * S

        xb = jnp.clip(x0, 0.0, FW - 2)
        yb = jnp.clip(y0, 0.0, FH - 2)
        pidx = base + yb.astype(jnp.int32) * FW + xb.astype(jnp.int32)
        actrep = act.astype(jnp.int32) + jnp.zeros((BQ, 64), jnp.int32)
        idx_ref[...] = jnp.concatenate([pidx, actrep], axis=1)

        w_parts = []
        for dy in (0, 1):
            for dx in (0, 1):
                xs = xb + dx
                ys = yb + dy
                wx = (jnp.where(xs == x0, 1.0 - fx, 0.0)
                      + jnp.where(xs == x0 + 1.0, fx, 0.0))
                wy = (jnp.where(ys == y0, 1.0 - fy, 0.0)
                      + jnp.where(ys == y0 + 1.0, fy, 0.0))
                w_parts.append(awa * wx * wy)
        w_ref[...] = jnp.concatenate(w_parts, axis=1)

    return pl.pallas_call(
        body,
        grid=(N_CAM, NQB),
        in_specs=[
            pl.BlockSpec((1, BQ, D), lambda n, qb: (n, qb, 0)),
            pl.BlockSpec((1, BQ, D), lambda n, qb: (n, qb, 0)),
            pl.BlockSpec((1, BQ, P), lambda n, qb: (n, qb, 0)),
            pl.BlockSpec((1, BQ, P), lambda n, qb: (n, qb, 0)),
            pl.BlockSpec((D, 2 * H * P), lambda n, qb: (0, 0)),
            pl.BlockSpec((1, 2 * H * P), lambda n, qb: (0, 0)),
            pl.BlockSpec((D, H * P), lambda n, qb: (0, 0)),
            pl.BlockSpec((1, H * P), lambda n, qb: (0, 0)),
        ],
        out_specs=[
            pl.BlockSpec((BQ, 2 * NPTS), lambda n, qb: (n * NQB + qb, 0)),
            pl.BlockSpec((BQ, NSLOT), lambda n, qb: (n * NQB + qb, 0)),
        ],
        out_shape=[
            jax.ShapeDtypeStruct((ROWS, 2 * NPTS), jnp.int32),
            jax.ShapeDtypeStruct((ROWS, NSLOT), jnp.float32),
        ],
    )(q2, p2, rpt2, bm2, wso_t, bso2, waw_t, baw2)


def _broadcast_lane(vec16, lane):
    idx = jnp.full((16, 1), lane, dtype=jnp.int32)
    dn = lax.GatherDimensionNumbers(offset_dims=(), collapsed_slice_dims=(0,),
                                    start_index_map=(0,))
    return lax.gather(vec16, idx, dn, (1,),
                      mode=lax.GatherScatterMode.PROMISE_IN_BOUNDS)


def _sc_gather_accum(table, idx2, w2):
    mesh = plsc.VectorSubcoreMesh(core_axis_name="c", subcore_axis_name="s")

    @functools.partial(
        pl.kernel,
        mesh=mesh,
        out_type=jax.ShapeDtypeStruct((ROWS, NSLOT), jnp.float32),
        scratch_types=[
            pltpu.VMEM((CH, 2 * NPTS), jnp.int32),
            pltpu.VMEM((2 * CH, 128), jnp.float32),
            pltpu.VMEM((GSLOTS, 4 * DH), jnp.float32),
            pltpu.VMEM((CH, NSLOT), jnp.float32),
            pltpu.SemaphoreType.DMA,
        ],
    )
    def sc_kernel(table_hbm, idx_hbm, w_hbm, out_hbm, idx_v, w_v, rows_v,
                  out_v, gsem):
        wid = lax.axis_index("s") * 2 + lax.axis_index("c")
        nbase = NCHUNK // NW
        nch = nbase + (wid < NCHUNK - nbase * NW).astype(jnp.int32)

        def row_flag(j):
            return idx_v[j, pl.ds(NPTS, 16)][0] > 0

        def gather_descr(j):
            return pltpu.make_async_copy(
                table_hbm.at[idx_v.at[j, pl.ds(0, NPTS)]],
                rows_v.at[pl.ds(j * NPTS, NPTS)], gsem)

        def chunk_body(i, _):
            c = wid + i * NW
            base = c * CH
            pltpu.sync_copy(idx_hbm.at[pl.ds(base, CH), :], idx_v)
            pltpu.sync_copy(w_hbm.at[pl.ds(2 * base, 2 * CH), :], w_v)
            for j in range(CH):
                @pl.when(row_flag(j))
                def _():
                    gather_descr(j).start()
            for j in range(CH):
                @pl.when(row_flag(j))
                def _():
                    gather_descr(j).wait()

            def row_body(r, _):
                flag = idx_v[r, pl.ds(NPTS, 16)][0] > 0

                @pl.when(flag)
                def _():
                    for h in range(H):
                        acc0 = jnp.zeros((16,), jnp.float32)
                        acc1 = jnp.zeros((16,), jnp.float32)
                        for sl in range(4):
                            wrow = 2 * r + (sl // 2)
                            astart = (sl % 2) * 64 + (h // 2) * 16
                            wreg = w_v[wrow, pl.ds(astart, 16)]
                            lane0 = (h % 2) * 8
                            for p in range(P):
                                wb = _broadcast_lane(wreg, lane0 + p)
                                g_row = r * NPTS + h * P + p
                                acc0 = acc0 + wb * rows_v[g_row,
                                                          pl.ds(sl * DH, 16)]
                                acc1 = acc1 + wb * rows_v[
                                    g_row, pl.ds(sl * DH + 16, 16)]
                        out_v[r, pl.ds(h * DH, 16)] = acc0
                        out_v[r, pl.ds(h * DH + 16, 16)] = acc1

                @pl.when(jnp.logical_not(flag))
                def _():
                    z = jnp.zeros((16,), jnp.float32)
                    for k in range(NSLOT // 16):
                        out_v[r, pl.ds(k * 16, 16)] = z
                return 0

            lax.fori_loop(0, CH, row_body, 0)
            pltpu.sync_copy(out_v, out_hbm.at[pl.ds(base, CH), :])
            return 0

        lax.fori_loop(0, nch, chunk_body, 0)

    return sc_kernel(table, idx2, w2)


def _tc_reduce_project(sc_out3, bm2, wout_t, bout2):
    def body(s_ref, bm_ref, w_ref, b_ref, o_ref):
        acc = s_ref[0]
        for i in range(1, N_CAM):
            acc = acc + s_ref[i]
        zi = lax.broadcasted_iota(jnp.int32, (P, 1), 0)
        selz = (zi % 2 == 0).astype(jnp.float32)
        cnt = jnp.zeros((BQ, 1), jnp.float32)
        for i in range(N_CAM):
            cnt = cnt + (lax.dot_general(
                bm_ref[i], selz, (((1,), (0,)), ((), ())),
                preferred_element_type=jnp.float32,
                            precision=lax.Precision.HIGHEST) > 0).astype(jnp.float32)
        slots = acc / jnp.maximum(cnt, 1.0)
        o_ref[0] = lax.dot_general(slots, w_ref[...], (((1,), (0,)), ((), ())),
                                   preferred_element_type=jnp.float32,
                            precision=lax.Precision.HIGHEST) + b_ref[...]

    return pl.pallas_call(
        body,
        grid=(NQB,),
        in_specs=[
            pl.BlockSpec((N_CAM, BQ, D), lambda qb: (0, qb, 0)),
            pl.BlockSpec((N_CAM, BQ, P), lambda qb: (0, qb, 0)),
            pl.BlockSpec((D, D), lambda qb: (0, 0)),
            pl.BlockSpec((1, D), lambda qb: (0, 0)),
        ],
        out_specs=pl.BlockSpec((1, BQ, D), lambda qb: (0, qb, 0)),
        out_shape=jax.ShapeDtypeStruct((1, Q_LEN, D), jnp.float32),
    )(sc_out3, bm2, wout_t, bout2)


def kernel(queries, pos_emb, lvl_emb, cam_emb, feat0, reference_points_3D,
           bev_mask, W_so, b_so, W_aw, b_aw, W_v, b_v, W_out, b_out):
    feat2 = feat0.reshape(N_CAM, D, S)
    cam3 = cam_emb.reshape(N_CAM, 1, D)
    wv3 = W_v.reshape(H, DH, D)
    bv3 = b_v.reshape(H, 1, DH)
    table = _tc_value_table(feat2, lvl_emb, cam3, wv3, bv3)
    table = table.reshape(N_CAM * H * S, 4 * DH)

    W_so_x = W_so[0::2]
    W_so_y = W_so[1::2]
    wso_t = jnp.concatenate([W_so_x, W_so_y], axis=0).T
    bso2 = jnp.concatenate([b_so[0::2], b_so[1::2]])[None, :]
    waw_t = W_aw.T
    baw2 = b_aw[None, :]

    q2 = queries.reshape(N_CAM, Q_LEN, D)
    p2 = pos_emb.reshape(N_CAM, Q_LEN, D)
    rpt2 = reference_points_3D.reshape(N_CAM, Q_LEN, NUM_Z * 2)
    bm2 = bev_mask.reshape(N_CAM, Q_LEN, NUM_Z * 2).astype(jnp.float32)

    idx, w = _tc_index_weights(q2, p2, rpt2, bm2, wso_t, bso2, waw_t, baw2)

    sc_out = _sc_gather_accum(table, idx, w.reshape(ROWS * 2, 128))
    sc_out3 = sc_out.reshape(N_CAM, Q_LEN, D)

    return _tc_reduce_project(sc_out3, bm2, W_out.T, b_out[None, :])


# speedup vs baseline: 135.9195x; 1.1739x over previous
import functools

import jax
import jax.numpy as jnp
from jax import lax
from jax.experimental import pallas as pl
from jax.experimental.pallas import tpu as pltpu
from jax.experimental.pallas import tpu_sc as plsc

N_CAM = 6
Q_LEN = 10000
D = 256
H = 8
DH = 32
P = 8
NUM_Z = 4
FH = 60
FW = 100
S = FH * FW
ROWS = N_CAM * Q_LEN
NSLOT = H * P * 4

NPTS = H * P
NW = 32
CH = 4
NCHUNK = ROWS // CH
GSLOTS = CH * NPTS


def _tc_value_table(feat2, lvl_emb, cam_emb, wv3, bv2):
    def body(f_ref, lvl_ref, cam_ref, wv_ref, bv_ref, out_ref):
        f = f_ref[0]
        wv = wv_ref[0]
        t = lax.dot_general(f, wv, (((0,), (1,)), ((), ())),
                            preferred_element_type=jnp.float32,
                            precision=lax.Precision.HIGHEST)
        lc = lvl_ref[...] + cam_ref[0]
        rb = lax.dot_general(lc, wv, (((1,), (1,)), ((), ())),
                             preferred_element_type=jnp.float32,
                            precision=lax.Precision.HIGHEST)
        t = t + rb + bv_ref[0]
        z = jnp.zeros((101, DH), jnp.float32)
        t1 = jnp.concatenate([t[1:], z[:1]], axis=0)
        t100 = jnp.concatenate([t[FW:], z[:FW]], axis=0)
        t101 = jnp.concatenate([t[FW + 1:], z], axis=0)
        out_ref[0] = jnp.concatenate([t, t1, t100, t101], axis=1)

    return pl.pallas_call(
        body,
        grid=(N_CAM, H),
        in_specs=[
            pl.BlockSpec((1, D, S), lambda n, h: (n, 0, 0)),
            pl.BlockSpec((1, D), lambda n, h: (0, 0)),
            pl.BlockSpec((1, 1, D), lambda n, h: (n, 0, 0)),
            pl.BlockSpec((1, DH, D), lambda n, h: (h, 0, 0)),
            pl.BlockSpec((1, 1, DH), lambda n, h: (h, 0, 0)),
        ],
        out_specs=pl.BlockSpec((1, S, 4 * DH), lambda n, h: (n * H + h, 0, 0)),
        out_shape=jax.ShapeDtypeStruct((N_CAM * H, S, 4 * DH), jnp.float32),
    )(feat2, lvl_emb, cam_emb, wv3, bv2)


BQ = 1000
NQB = Q_LEN // BQ


def _tc_index_weights(q2, p2, rpt2, bm2, wso_t, bso2, waw_t, baw2):
    def body(q_ref, p_ref, rpt_ref, bm_ref, wso_ref, bso_ref, waw_ref,
             baw_ref, idx_ref, w_ref):
        n = pl.program_id(0)
        qr = q_ref[0] + p_ref[0]
        so = lax.dot_general(qr, wso_ref[...], (((1,), (0,)), ((), ())),
                             preferred_element_type=jnp.float32,
                            precision=lax.Precision.HIGHEST) + bso_ref[...]
        awl = lax.dot_general(qr, waw_ref[...], (((1,), (0,)), ((), ())),
                              preferred_element_type=jnp.float32,
                            precision=lax.Precision.HIGHEST) + baw_ref[...]
        m = jnp.max(awl, axis=1, keepdims=True)
        e = jnp.exp(awl - m)
        gi = lax.broadcasted_iota(jnp.int32, (64, 64), 0) // P
        gj = lax.broadcasted_iota(jnp.int32, (64, 64), 1) // P
        G = (gi == gj).astype(jnp.float32)
        ssum = lax.dot_general(e, G, (((1,), (0,)), ((), ())),
                               preferred_element_type=jnp.float32,
                            precision=lax.Precision.HIGHEST)
        aw = e / ssum

        zi = lax.broadcasted_iota(jnp.int32, (P, 1), 0)
        selz = (zi % 2 == 0).astype(jnp.float32)
        act = (lax.dot_general(bm_ref[0], selz, (((1,), (0,)), ((), ())),
                               preferred_element_type=jnp.float32,
                            precision=lax.Precision.HIGHEST)
               > 0).astype(jnp.float32)
        awa = aw * act

        rj = lax.broadcasted_iota(jnp.int32, (P, 64), 0)
        cz = lax.broadcasted_iota(jnp.int32, (P, 64), 1) % NUM_Z
        Sx = (rj == 2 * cz).astype(jnp.float32)
        Sy = (rj == 2 * cz + 1).astype(jnp.float32)
        rx = lax.dot_general(rpt_ref[0], Sx, (((1,), (0,)), ((), ())),
                             preferred_element_type=jnp.float32,
                            precision=lax.Precision.HIGHEST)
        ry = lax.dot_general(rpt_ref[0], Sy, (((1,), (0,)), ((), ())),
                             preferred_element_type=jnp.float32,
                            precision=lax.Precision.HIGHEST)

        sox = so[:, :64]
        soy = so[:, 64:]
        x = rx * FW + sox - 0.5
        y = ry * FH + soy - 0.5
        x0 = jnp.floor(x)
        y0 = jnp.floor(y)
        fx = x - x0
        fy = y - y0

        hcol = lax.broadcasted_iota(jnp.int32, (BQ, 64), 1) // P
        base = n * (H * S) + hcol * S

        xb = jnp.clip(x0, 0.0, FW - 2)
        yb = jnp.clip(y0, 0.0, FH - 2)
        pidx = base + yb.astype(jnp.int32) * FW + xb.astype(jnp.int32)
        actrep = act.astype(jnp.int32) + jnp.zeros((BQ, 64), jnp.int32)
        idx_ref[...] = jnp.concatenate([pidx, actrep], axis=1)

        w_parts = []
        for dy in (0, 1):
            for dx in (0, 1):
                xs = xb + dx
                ys = yb + dy
                wx = (jnp.where(xs == x0, 1.0 - fx, 0.0)
                      + jnp.where(xs == x0 + 1.0, fx, 0.0))
                wy = (jnp.where(ys == y0, 1.0 - fy, 0.0)
                      + jnp.where(ys == y0 + 1.0, fy, 0.0))
                w_parts.append(awa * wx * wy)
        w_ref[...] = jnp.concatenate(w_parts, axis=1)

    return pl.pallas_call(
        body,
        grid=(N_CAM, NQB),
        in_specs=[
            pl.BlockSpec((1, BQ, D), lambda n, qb: (n, qb, 0)),
            pl.BlockSpec((1, BQ, D), lambda n, qb: (n, qb, 0)),
            pl.BlockSpec((1, BQ, P), lambda n, qb: (n, qb, 0)),
            pl.BlockSpec((1, BQ, P), lambda n, qb: (n, qb, 0)),
            pl.BlockSpec((D, 2 * H * P), lambda n, qb: (0, 0)),
            pl.BlockSpec((1, 2 * H * P), lambda n, qb: (0, 0)),
            pl.BlockSpec((D, H * P), lambda n, qb: (0, 0)),
            pl.BlockSpec((1, H * P), lambda n, qb: (0, 0)),
        ],
        out_specs=[
            pl.BlockSpec((BQ, 2 * NPTS), lambda n, qb: (n * NQB + qb, 0)),
            pl.BlockSpec((BQ, NSLOT), lambda n, qb: (n * NQB + qb, 0)),
        ],
        out_shape=[
            jax.ShapeDtypeStruct((ROWS, 2 * NPTS), jnp.int32),
            jax.ShapeDtypeStruct((ROWS, NSLOT), jnp.float32),
        ],
    )(q2, p2, rpt2, bm2, wso_t, bso2, waw_t, baw2)


def _broadcast_lane(vec16, lane):
    idx = jnp.full((16, 1), lane, dtype=jnp.int32)
    dn = lax.GatherDimensionNumbers(offset_dims=(), collapsed_slice_dims=(0,),
                                    start_index_map=(0,))
    return lax.gather(vec16, idx, dn, (1,),
                      mode=lax.GatherScatterMode.PROMISE_IN_BOUNDS)


def _sc_gather_accum(table, idx3, w3):
    mesh = plsc.VectorSubcoreMesh(core_axis_name="c", subcore_axis_name="s")

    @functools.partial(
        pl.kernel,
        mesh=mesh,
        out_type=jax.ShapeDtypeStruct((NCHUNK, CH, NSLOT), jnp.float32),
        scratch_types=[
            pltpu.VMEM((2, CH, 2 * NPTS), jnp.int32),
            pltpu.VMEM((2, 2 * CH, 128), jnp.float32),
            pltpu.VMEM((2, GSLOTS, 4 * DH), jnp.float32),
            pltpu.VMEM((2, CH, NSLOT), jnp.float32),
            pltpu.SemaphoreType.DMA,
            pltpu.SemaphoreType.DMA,
            pltpu.SemaphoreType.DMA,
            pltpu.SemaphoreType.DMA,
        ],
    )
    def sc_kernel(table_hbm, idx_hbm, w_hbm, out_hbm, idx_v, w_v, rows_v,
                  out_v, sem_g, sem_i, sem_w, sem_o):
        wid = lax.axis_index("s") * 2 + lax.axis_index("c")
        nbase = NCHUNK // NW
        nch = nbase + (wid < NCHUNK - nbase * NW).astype(jnp.int32)

        def chunk_of(i):
            return wid + i * NW

        def io_descrs(i, b):
            c = chunk_of(i)
            return (pltpu.make_async_copy(idx_hbm.at[c], idx_v.at[b], sem_i),
                    pltpu.make_async_copy(w_hbm.at[c], w_v.at[b], sem_w))

        def out_descr(i, b):
            return pltpu.make_async_copy(out_v.at[b], out_hbm.at[chunk_of(i)],
                                         sem_o)

        def row_flag(b, j):
            return idx_v[b, j, pl.ds(NPTS, 16)][0] > 0

        def gather_descr(b, j):
            return pltpu.make_async_copy(
                table_hbm.at[idx_v.at[b, j, pl.ds(0, NPTS)]],
                rows_v.at[b, pl.ds(j * NPTS, NPTS)], sem_g)

        def fire_gathers(b):
            for j in range(CH):
                @pl.when(row_flag(b, j))
                def _():
                    gather_descr(b, j).start()

        def wait_gathers(b):
            for j in range(CH):
                @pl.when(row_flag(b, j))
                def _():
                    gather_descr(b, j).wait()

        d0i, d0w = io_descrs(0, 0)
        d0i.start()
        d0w.start()
        d0i.wait()
        d0w.wait()
        fire_gathers(0)
        d1i, d1w = io_descrs(1, 1)
        d1i.start()
        d1w.start()

        def chunk_body(i, _):
            b = lax.rem(i, 2)
            nb = 1 - b
            wait_gathers(b)

            @pl.when(i + 1 < nch)
            def _():
                di, dw = io_descrs(i + 1, nb)
                di.wait()
                dw.wait()
                fire_gathers(nb)

            @pl.when(i >= 2)
            def _():
                out_descr(i - 2, b).wait()

            def row_body(r, _):
                flag = idx_v[b, r, pl.ds(NPTS, 16)][0] > 0

                @pl.when(flag)
                def _():
                    for h in range(H):
                        acc0 = jnp.zeros((16,), jnp.float32)
                        acc1 = jnp.zeros((16,), jnp.float32)
                        for sl in range(4):
                            wrow = 2 * r + (sl // 2)
                            astart = (sl % 2) * 64 + (h // 2) * 16
                            wreg = w_v[b, wrow, pl.ds(astart, 16)]
                            lane0 = (h % 2) * 8
                            for p in range(P):
                                wb = _broadcast_lane(wreg, lane0 + p)
                                g_row = r * NPTS + h * P + p
                                acc0 = acc0 + wb * rows_v[
                                    b, g_row, pl.ds(sl * DH, 16)]
                                acc1 = acc1 + wb * rows_v[
                                    b, g_row, pl.ds(sl * DH + 16, 16)]
                        out_v[b, r, pl.ds(h * DH, 16)] = acc0
                        out_v[b, r, pl.ds(h * DH + 16, 16)] = acc1

                @pl.when(jnp.logical_not(flag))
                def _():
                    z = jnp.zeros((16,), jnp.float32)
                    for k in range(NSLOT // 16):
                        out_v[b, r, pl.ds(k * 16, 16)] = z
                return 0

            lax.fori_loop(0, CH, row_body, 0)
            out_descr(i, b).start()

            @pl.when(i + 2 < nch)
            def _():
                di, dw = io_descrs(i + 2, b)
                di.start()
                dw.start()
            return 0

        lax.fori_loop(0, nch, chunk_body, 0)
        out_descr(nch - 2, lax.rem(nch - 2, 2)).wait()
        out_descr(nch - 1, lax.rem(nch - 1, 2)).wait()

    return sc_kernel(table, idx3, w3)


def _tc_reduce_project(sc_out3, bm2, wout_t, bout2):
    def body(s_ref, bm_ref, w_ref, b_ref, o_ref):
        acc = s_ref[0]
        for i in range(1, N_CAM):
            acc = acc + s_ref[i]
        zi = lax.broadcasted_iota(jnp.int32, (P, 1), 0)
        selz = (zi % 2 == 0).astype(jnp.float32)
        cnt = jnp.zeros((BQ, 1), jnp.float32)
        for i in range(N_CAM):
            cnt = cnt + (lax.dot_general(
                bm_ref[i], selz, (((1,), (0,)), ((), ())),
                preferred_element_type=jnp.float32,
                            precision=lax.Precision.HIGHEST) > 0).astype(jnp.float32)
        slots = acc / jnp.maximum(cnt, 1.0)
        o_ref[0] = lax.dot_general(slots, w_ref[...], (((1,), (0,)), ((), ())),
                                   preferred_element_type=jnp.float32,
                            precision=lax.Precision.HIGHEST) + b_ref[...]

    return pl.pallas_call(
        body,
        grid=(NQB,),
        in_specs=[
            pl.BlockSpec((N_CAM, BQ, D), lambda qb: (0, qb, 0)),
            pl.BlockSpec((N_CAM, BQ, P), lambda qb: (0, qb, 0)),
            pl.BlockSpec((D, D), lambda qb: (0, 0)),
            pl.BlockSpec((1, D), lambda qb: (0, 0)),
        ],
        out_specs=pl.BlockSpec((1, BQ, D), lambda qb: (0, qb, 0)),
        out_shape=jax.ShapeDtypeStruct((1, Q_LEN, D), jnp.float32),
    )(sc_out3, bm2, wout_t, bout2)


def kernel(queries, pos_emb, lvl_emb, cam_emb, feat0, reference_points_3D,
           bev_mask, W_so, b_so, W_aw, b_aw, W_v, b_v, W_out, b_out):
    feat2 = feat0.reshape(N_CAM, D, S)
    cam3 = cam_emb.reshape(N_CAM, 1, D)
    wv3 = W_v.reshape(H, DH, D)
    bv3 = b_v.reshape(H, 1, DH)
    table = _tc_value_table(feat2, lvl_emb, cam3, wv3, bv3)
    table = table.reshape(N_CAM * H * S, 4 * DH)

    W_so_x = W_so[0::2]
    W_so_y = W_so[1::2]
    wso_t = jnp.concatenate([W_so_x, W_so_y], axis=0).T
    bso2 = jnp.concatenate([b_so[0::2], b_so[1::2]])[None, :]
    waw_t = W_aw.T
    baw2 = b_aw[None, :]

    q2 = queries.reshape(N_CAM, Q_LEN, D)
    p2 = pos_emb.reshape(N_CAM, Q_LEN, D)
    rpt2 = reference_points_3D.reshape(N_CAM, Q_LEN, NUM_Z * 2)
    bm2 = bev_mask.reshape(N_CAM, Q_LEN, NUM_Z * 2).astype(jnp.float32)

    idx, w = _tc_index_weights(q2, p2, rpt2, bm2, wso_t, bso2, waw_t, baw2)

    sc_out = _sc_gather_accum(table, idx.reshape(NCHUNK, CH, 2 * NPTS),
                              w.reshape(NCHUNK, 2 * CH, 128))
    sc_out3 = sc_out.reshape(N_CAM, Q_LEN, D)

    return _tc_reduce_project(sc_out3, bm2, W_out.T, b_out[None, :])


# speedup vs baseline: 140.7339x; 1.0354x over previous
import functools

import jax
import jax.numpy as jnp
from jax import lax
from jax.experimental import pallas as pl
from jax.experimental.pallas import tpu as pltpu
from jax.experimental.pallas import tpu_sc as plsc

N_CAM = 6
Q_LEN = 10000
D = 256
H = 8
DH = 32
P = 8
NUM_Z = 4
FH = 60
FW = 100
S = FH * FW
ROWS = N_CAM * Q_LEN
NSLOT = H * P * 4

NPTS = H * P
NW = 32
CH = 6
NCHUNK = ROWS // CH
GSLOTS = CH * NPTS


def _tc_value_table(feat2, lvl_emb, cam_emb, wv3, bv2):
    def body(f_ref, lvl_ref, cam_ref, wv_ref, bv_ref, out_ref):
        f = f_ref[0]
        wv = wv_ref[0]
        t = lax.dot_general(f, wv, (((0,), (1,)), ((), ())),
                            preferred_element_type=jnp.float32,
                            precision=lax.Precision.HIGHEST)
        lc = lvl_ref[...] + cam_ref[0]
        rb = lax.dot_general(lc, wv, (((1,), (1,)), ((), ())),
                             preferred_element_type=jnp.float32,
                            precision=lax.Precision.HIGHEST)
        t = t + rb + bv_ref[0]
        z = jnp.zeros((101, DH), jnp.float32)
        t1 = jnp.concatenate([t[1:], z[:1]], axis=0)
        t100 = jnp.concatenate([t[FW:], z[:FW]], axis=0)
        t101 = jnp.concatenate([t[FW + 1:], z], axis=0)
        out_ref[0] = jnp.concatenate([t, t1, t100, t101], axis=1)

    return pl.pallas_call(
        body,
        grid=(N_CAM, H),
        in_specs=[
            pl.BlockSpec((1, D, S), lambda n, h: (n, 0, 0)),
            pl.BlockSpec((1, D), lambda n, h: (0, 0)),
            pl.BlockSpec((1, 1, D), lambda n, h: (n, 0, 0)),
            pl.BlockSpec((1, DH, D), lambda n, h: (h, 0, 0)),
            pl.BlockSpec((1, 1, DH), lambda n, h: (h, 0, 0)),
        ],
        out_specs=pl.BlockSpec((1, S, 4 * DH), lambda n, h: (n * H + h, 0, 0)),
        out_shape=jax.ShapeDtypeStruct((N_CAM * H, S, 4 * DH), jnp.float32),
    )(feat2, lvl_emb, cam_emb, wv3, bv2)


BQ = 1000
NQB = Q_LEN // BQ


def _tc_index_weights(q2, p2, rpt2, bm2, wso_t, bso2, waw_t, baw2):
    def body(q_ref, p_ref, rpt_ref, bm_ref, wso_ref, bso_ref, waw_ref,
             baw_ref, idx_ref, w_ref):
        n = pl.program_id(0)
        qr = q_ref[0] + p_ref[0]
        so = lax.dot_general(qr, wso_ref[...], (((1,), (0,)), ((), ())),
                             preferred_element_type=jnp.float32,
                            precision=lax.Precision.HIGHEST) + bso_ref[...]
        awl = lax.dot_general(qr, waw_ref[...], (((1,), (0,)), ((), ())),
                              preferred_element_type=jnp.float32,
                            precision=lax.Precision.HIGHEST) + baw_ref[...]
        m = jnp.max(awl, axis=1, keepdims=True)
        e = jnp.exp(awl - m)
        gi = lax.broadcasted_iota(jnp.int32, (64, 64), 0) // P
        gj = lax.broadcasted_iota(jnp.int32, (64, 64), 1) // P
        G = (gi == gj).astype(jnp.float32)
        ssum = lax.dot_general(e, G, (((1,), (0,)), ((), ())),
                               preferred_element_type=jnp.float32,
                            precision=lax.Precision.HIGHEST)
        aw = e / ssum

        zi = lax.broadcasted_iota(jnp.int32, (P, 1), 0)
        selz = (zi % 2 == 0).astype(jnp.float32)
        act = (lax.dot_general(bm_ref[0], selz, (((1,), (0,)), ((), ())),
                               preferred_element_type=jnp.float32,
                            precision=lax.Precision.HIGHEST)
               > 0).astype(jnp.float32)
        awa = aw * act

        rj = lax.broadcasted_iota(jnp.int32, (P, 64), 0)
        cz = lax.broadcasted_iota(jnp.int32, (P, 64), 1) % NUM_Z
        Sx = (rj == 2 * cz).astype(jnp.float32)
        Sy = (rj == 2 * cz + 1).astype(jnp.float32)
        rx = lax.dot_general(rpt_ref[0], Sx, (((1,), (0,)), ((), ())),
                             preferred_element_type=jnp.float32,
                            precision=lax.Precision.HIGHEST)
        ry = lax.dot_general(rpt_ref[0], Sy, (((1,), (0,)), ((), ())),
                             preferred_element_type=jnp.float32,
                            precision=lax.Precision.HIGHEST)

        sox = so[:, :64]
        soy = so[:, 64:]
        x = rx * FW + sox - 0.5
        y = ry * FH + soy - 0.5
        x0 = jnp.floor(x)
        y0 = jnp.floor(y)
        fx = x - x0
        fy = y - y0

        hcol = lax.broadcasted_iota(jnp.int32, (BQ, 64), 1) // P
        base = n * (H * S) + hcol * S

        xb = jnp.clip(x0, 0.0, FW - 2)
        yb = jnp.clip(y0, 0.0, FH - 2)
        pidx = base + yb.astype(jnp.int32) * FW + xb.astype(jnp.int32)
        actrep = act.astype(jnp.int32) + jnp.zeros((BQ, 64), jnp.int32)
        idx_ref[...] = jnp.concatenate([pidx, actrep], axis=1)

        w_parts = []
        for dy in (0, 1):
            for dx in (0, 1):
                xs = xb + dx
                ys = yb + dy
                wx = (jnp.where(xs == x0, 1.0 - fx, 0.0)
                      + jnp.where(xs == x0 + 1.0, fx, 0.0))
                wy = (jnp.where(ys == y0, 1.0 - fy, 0.0)
                      + jnp.where(ys == y0 + 1.0, fy, 0.0))
                w_parts.append(awa * wx * wy)
        w_ref[...] = jnp.concatenate(w_parts, axis=1)

    return pl.pallas_call(
        body,
        grid=(N_CAM, NQB),
        in_specs=[
            pl.BlockSpec((1, BQ, D), lambda n, qb: (n, qb, 0)),
            pl.BlockSpec((1, BQ, D), lambda n, qb: (n, qb, 0)),
            pl.BlockSpec((1, BQ, P), lambda n, qb: (n, qb, 0)),
            pl.BlockSpec((1, BQ, P), lambda n, qb: (n, qb, 0)),
            pl.BlockSpec((D, 2 * H * P), lambda n, qb: (0, 0)),
            pl.BlockSpec((1, 2 * H * P), lambda n, qb: (0, 0)),
            pl.BlockSpec((D, H * P), lambda n, qb: (0, 0)),
            pl.BlockSpec((1, H * P), lambda n, qb: (0, 0)),
        ],
        out_specs=[
            pl.BlockSpec((BQ, 2 * NPTS), lambda n, qb: (n * NQB + qb, 0)),
            pl.BlockSpec((BQ, NSLOT), lambda n, qb: (n * NQB + qb, 0)),
        ],
        out_shape=[
            jax.ShapeDtypeStruct((ROWS, 2 * NPTS), jnp.int32),
            jax.ShapeDtypeStruct((ROWS, NSLOT), jnp.float32),
        ],
    )(q2, p2, rpt2, bm2, wso_t, bso2, waw_t, baw2)


def _broadcast_lane(vec16, lane):
    idx = jnp.full((16, 1), lane, dtype=jnp.int32)
    dn = lax.GatherDimensionNumbers(offset_dims=(), collapsed_slice_dims=(0,),
                                    start_index_map=(0,))
    return lax.gather(vec16, idx, dn, (1,),
                      mode=lax.GatherScatterMode.PROMISE_IN_BOUNDS)


def _sc_gather_accum(table, idx3, w3):
    mesh = plsc.VectorSubcoreMesh(core_axis_name="c", subcore_axis_name="s")

    @functools.partial(
        pl.kernel,
        mesh=mesh,
        out_type=jax.ShapeDtypeStruct((NCHUNK, CH, NSLOT), jnp.float32),
        scratch_types=[
            pltpu.VMEM((2, CH, 2 * NPTS), jnp.int32),
            pltpu.VMEM((2, 2 * CH, 128), jnp.float32),
            pltpu.VMEM((2, GSLOTS, 4 * DH), jnp.float32),
            pltpu.VMEM((2, CH, NSLOT), jnp.float32),
            pltpu.SemaphoreType.DMA,
            pltpu.SemaphoreType.DMA,
            pltpu.SemaphoreType.DMA,
            pltpu.SemaphoreType.DMA,
        ],
    )
    def sc_kernel(table_hbm, idx_hbm, w_hbm, out_hbm, idx_v, w_v, rows_v,
                  out_v, sem_g, sem_i, sem_w, sem_o):
        wid = lax.axis_index("s") * 2 + lax.axis_index("c")
        nbase = NCHUNK // NW
        nch = nbase + (wid < NCHUNK - nbase * NW).astype(jnp.int32)

        def chunk_of(i):
            return wid + i * NW

        def io_descrs(i, b):
            c = chunk_of(i)
            return (pltpu.make_async_copy(idx_hbm.at[c], idx_v.at[b], sem_i),
                    pltpu.make_async_copy(w_hbm.at[c], w_v.at[b], sem_w))

        def out_descr(i, b):
            return pltpu.make_async_copy(out_v.at[b], out_hbm.at[chunk_of(i)],
                                         sem_o)

        def row_flag(b, j):
            return idx_v[b, j, pl.ds(NPTS, 16)][0] > 0

        def gather_descr(b, j):
            return pltpu.make_async_copy(
                table_hbm.at[idx_v.at[b, j, pl.ds(0, NPTS)]],
                rows_v.at[b, pl.ds(j * NPTS, NPTS)], sem_g)

        def fire_gathers(b):
            for j in range(CH):
                @pl.when(row_flag(b, j))
                def _():
                    gather_descr(b, j).start()

        def wait_gathers(b):
            for j in range(CH):
                @pl.when(row_flag(b, j))
                def _():
                    gather_descr(b, j).wait()

        d0i, d0w = io_descrs(0, 0)
        d0i.start()
        d0w.start()
        d0i.wait()
        d0w.wait()
        fire_gathers(0)
        d1i, d1w = io_descrs(1, 1)
        d1i.start()
        d1w.start()

        def chunk_body(i, _):
            b = lax.rem(i, 2)
            nb = 1 - b
            wait_gathers(b)

            @pl.when(i + 1 < nch)
            def _():
                di, dw = io_descrs(i + 1, nb)
                di.wait()
                dw.wait()
                fire_gathers(nb)

            @pl.when(i >= 2)
            def _():
                out_descr(i - 2, b).wait()

            def row_body(r, _):
                flag = idx_v[b, r, pl.ds(NPTS, 16)][0] > 0

                @pl.when(flag)
                def _():
                    for h in range(H):
                        acc0 = jnp.zeros((16,), jnp.float32)
                        acc1 = jnp.zeros((16,), jnp.float32)
                        for sl in range(4):
                            wrow = 2 * r + (sl // 2)
                            astart = (sl % 2) * 64 + (h // 2) * 16
                            wreg = w_v[b, wrow, pl.ds(astart, 16)]
                            lane0 = (h % 2) * 8
                            for p in range(P):
                                wb = _broadcast_lane(wreg, lane0 + p)
                                g_row = r * NPTS + h * P + p
                                acc0 = acc0 + wb * rows_v[
                                    b, g_row, pl.ds(sl * DH, 16)]
                                acc1 = acc1 + wb * rows_v[
                                    b, g_row, pl.ds(sl * DH + 16, 16)]
                        out_v[b, r, pl.ds(h * DH, 16)] = acc0
                        out_v[b, r, pl.ds(h * DH + 16, 16)] = acc1

                @pl.when(jnp.logical_not(flag))
                def _():
                    z = jnp.zeros((16,), jnp.float32)
                    for k in range(NSLOT // 16):
                        out_v[b, r, pl.ds(k * 16, 16)] = z
                return 0

            lax.fori_loop(0, CH, row_body, 0)
            out_descr(i, b).start()

            @pl.when(i + 2 < nch)
            def _():
                di, dw = io_descrs(i + 2, b)
                di.start()
                dw.start()
            return 0

        lax.fori_loop(0, nch, chunk_body, 0)
        out_descr(nch - 2, lax.rem(nch - 2, 2)).wait()
        out_descr(nch - 1, lax.rem(nch - 1, 2)).wait()

    return sc_kernel(table, idx3, w3)


def _tc_reduce_project(sc_out3, bm2, wout_t, bout2):
    def body(s_ref, bm_ref, w_ref, b_ref, o_ref):
        acc = s_ref[0]
        for i in range(1, N_CAM):
            acc = acc + s_ref[i]
        zi = lax.broadcasted_iota(jnp.int32, (P, 1), 0)
        selz = (zi % 2 == 0).astype(jnp.float32)
        cnt = jnp.zeros((BQ, 1), jnp.float32)
        for i in range(N_CAM):
            cnt = cnt + (lax.dot_general(
                bm_ref[i], selz, (((1,), (0,)), ((), ())),
                preferred_element_type=jnp.float32,
                            precision=lax.Precision.HIGHEST) > 0).astype(jnp.float32)
        slots = acc / jnp.maximum(cnt, 1.0)
        o_ref[0] = lax.dot_general(slots, w_ref[...], (((1,), (0,)), ((), ())),
                                   preferred_element_type=jnp.float32,
                            precision=lax.Precision.HIGHEST) + b_ref[...]

    return pl.pallas_call(
        body,
        grid=(NQB,),
        in_specs=[
            pl.BlockSpec((N_CAM, BQ, D), lambda qb: (0, qb, 0)),
            pl.BlockSpec((N_CAM, BQ, P), lambda qb: (0, qb, 0)),
            pl.BlockSpec((D, D), lambda qb: (0, 0)),
            pl.BlockSpec((1, D), lambda qb: (0, 0)),
        ],
        out_specs=pl.BlockSpec((1, BQ, D), lambda qb: (0, qb, 0)),
        out_shape=jax.ShapeDtypeStruct((1, Q_LEN, D), jnp.float32),
    )(sc_out3, bm2, wout_t, bout2)


def kernel(queries, pos_emb, lvl_emb, cam_emb, feat0, reference_points_3D,
           bev_mask, W_so, b_so, W_aw, b_aw, W_v, b_v, W_out, b_out):
    feat2 = feat0.reshape(N_CAM, D, S)
    cam3 = cam_emb.reshape(N_CAM, 1, D)
    wv3 = W_v.reshape(H, DH, D)
    bv3 = b_v.reshape(H, 1, DH)
    table = _tc_value_table(feat2, lvl_emb, cam3, wv3, bv3)
    table = table.reshape(N_CAM * H * S, 4 * DH)

    W_so_x = W_so[0::2]
    W_so_y = W_so[1::2]
    wso_t = jnp.concatenate([W_so_x, W_so_y], axis=0).T
    bso2 = jnp.concatenate([b_so[0::2], b_so[1::2]])[None, :]
    waw_t = W_aw.T
    baw2 = b_aw[None, :]

    q2 = queries.reshape(N_CAM, Q_LEN, D)
    p2 = pos_emb.reshape(N_CAM, Q_LEN, D)
    rpt2 = reference_points_3D.reshape(N_CAM, Q_LEN, NUM_Z * 2)
    bm2 = bev_mask.reshape(N_CAM, Q_LEN, NUM_Z * 2).astype(jnp.float32)

    idx, w = _tc_index_weights(q2, p2, rpt2, bm2, wso_t, bso2, waw_t, baw2)

    sc_out = _sc_gather_accum(table, idx.reshape(NCHUNK, CH, 2 * NPTS),
                              w.reshape(NCHUNK, 2 * CH, 128))
    sc_out3 = sc_out.reshape(N_CAM, Q_LEN, D)

    return _tc_reduce_project(sc_out3, bm2, W_out.T, b_out[None, :])
